# Initial kernel scaffold; baseline (speedup 1.0000x reference)
import jax
import jax.numpy as jnp
from jax.experimental import pallas as pl


def kernel(s, o, p, W_words, W_rel, W_lin, b_lin):
    raise NotImplementedError("write your pallas kernel here")


# speedup vs baseline: 1.8038x; 1.8038x over previous
import functools

import jax
import jax.numpy as jnp
from jax import lax
from jax.experimental import pallas as pl
from jax.experimental.pallas import tpu as pltpu
from jax.experimental.pallas import tpu_sc as plsc

B = 4096
L = 20
DIM = 64
NC = 2
NS = 16
NW = NC * NS
RPW = B // NW
CR = 4
CI = CR * L
NCH = RPW // CR
NV = DIM // 16


def _sc_pool(s_r, o_r, p, W_words, W_rel):
    mesh = plsc.VectorSubcoreMesh(core_axis_name="c", subcore_axis_name="s")

    @functools.partial(
        pl.kernel,
        out_type=[jax.ShapeDtypeStruct((B, DIM), jnp.float32)] * 3,
        mesh=mesh,
        compiler_params=pltpu.CompilerParams(use_tc_tiling_on_sc=False),
        scratch_types=[
            pltpu.VMEM((NCH, CI), jnp.int32),
            pltpu.VMEM((CI, DIM), jnp.float32),
            pltpu.VMEM((CI, DIM), jnp.float32),
            pltpu.VMEM((RPW, DIM), jnp.float32),
            pltpu.VMEM((RPW,), jnp.int32),
            pltpu.VMEM((RPW, DIM), jnp.float32),
            pltpu.SemaphoreType.DMA,
            pltpu.SemaphoreType.DMA,
            pltpu.SemaphoreType.DMA,
        ],
    )
    def sc_kernel(s_hbm, o_hbm, p_hbm, ww_hbm, wr_hbm,
                  s_out, o_out, p_out,
                  idx_v, gbuf0, gbuf1, out_v, pidx_v, prow_v,
                  sem0, sem1, psem):
        wid = lax.axis_index("s") * NC + lax.axis_index("c")
        base = wid * RPW
        bufs = (gbuf0, gbuf1)
        sems = (sem0, sem1)

        pltpu.sync_copy(p_hbm.at[pl.ds(base, RPW)], pidx_v)
        pltpu.make_async_copy(wr_hbm.at[pidx_v], prow_v, psem).start()

        def run_side(side_hbm, side_out):
            pltpu.sync_copy(side_hbm.at[wid], idx_v)

            def start(c, b):
                pltpu.make_async_copy(ww_hbm.at[idx_v.at[c]], bufs[b],
                                      sems[b]).start()

            def wait(b):
                pltpu.make_async_copy(ww_hbm.at[idx_v.at[0]], bufs[b],
                                      sems[b]).wait()

            def reduce_chunk(c, gbuf):
                for r in range(CR):
                    accs = [gbuf[r * L, pl.ds(d * 16, 16)] for d in range(NV)]
                    for l in range(1, L):
                        for d in range(NV):
                            accs[d] = accs[d] + gbuf[r * L + l,
                                                     pl.ds(d * 16, 16)]
                    row = c * CR + r
                    for d in range(NV):
                        out_v[row, pl.ds(d * 16, 16)] = accs[d]

            start(0, 0)

            def body(g, carry):
                for b in range(2):
                    c = g * 2 + b
                    nxt = c + 1

                    @pl.when(nxt < NCH)
                    def _():
                        start(nxt, (b + 1) % 2)

                    wait(b)
                    reduce_chunk(c, bufs[b])
                return carry

            lax.fori_loop(0, NCH // 2, body, 0)
            pltpu.sync_copy(out_v, side_out.at[pl.ds(base, RPW)])

        run_side(s_hbm, s_out)
        run_side(o_hbm, o_out)

        pltpu.make_async_copy(wr_hbm.at[pidx_v], prow_v, psem).wait()
        pltpu.sync_copy(prow_v, p_out.at[pl.ds(base, RPW)])

    return sc_kernel(s_r, o_r, p, W_words, W_rel)


def _tc_dense(s, o, s_sum, o_sum, p_emb, W_lin_t, b_lin):
    def tc_kernel(s_ref, o_ref, ssum_ref, osum_ref, pemb_ref,
                  wlt_ref, bl_ref, out_ref):
        freq_s = jnp.sum((s_ref[...] != 0).astype(jnp.float32), axis=1,
                         keepdims=True)
        freq_o = jnp.sum((o_ref[...] != 0).astype(jnp.float32), axis=1,
                         keepdims=True)
        se = ssum_ref[...] * freq_s
        oe = osum_ref[...] * freq_o
        wlt = wlt_ref[...]
        bl = bl_ref[...]
        st = jnp.maximum(
            jnp.dot(se, wlt, preferred_element_type=jnp.float32) + bl, 0.0)
        ot = jnp.maximum(
            jnp.dot(oe, wlt, preferred_element_type=jnp.float32) + bl, 0.0)
        out_ref[...] = jnp.sum(st * pemb_ref[...] * ot, axis=1)

    return pl.pallas_call(
        tc_kernel,
        out_shape=jax.ShapeDtypeStruct((B,), jnp.float32),
    )(s, o, s_sum, o_sum, p_emb, W_lin_t, b_lin)


def kernel(s, o, p, W_words, W_rel, W_lin, b_lin):
    s_r = s.reshape(NW, NCH, CI).astype(jnp.int32)
    o_r = o.reshape(NW, NCH, CI).astype(jnp.int32)
    p_i = p.astype(jnp.int32)
    s_sum, o_sum, p_emb = _sc_pool(s_r, o_r, p_i, W_words, W_rel)
    bl = b_lin.reshape(1, DIM)
    return _tc_dense(s, o, s_sum, o_sum, p_emb, W_lin.T, bl)


# speedup vs baseline: 1.8996x; 1.0531x over previous
import functools

import jax
import jax.numpy as jnp
from jax import lax
from jax.experimental import pallas as pl
from jax.experimental.pallas import tpu as pltpu
from jax.experimental.pallas import tpu_sc as plsc

B = 4096
L = 20
DIM = 64
NC = 2
NS = 16
NW = NC * NS
RPW = B // NW
CR = 4
CI = CR * L
NCH = RPW // CR
NV = DIM // 16


def _sc_pool(s_r, o_r, p, W_words, W_rel):
    mesh = plsc.VectorSubcoreMesh(core_axis_name="c", subcore_axis_name="s")

    @functools.partial(
        pl.kernel,
        out_type=[jax.ShapeDtypeStruct((B, DIM), jnp.float32)] * 3,
        mesh=mesh,
        compiler_params=pltpu.CompilerParams(use_tc_tiling_on_sc=False),
        scratch_types=[
            pltpu.VMEM((NCH, CI), jnp.int32),
            pltpu.VMEM((CI, DIM), jnp.float32),
            pltpu.VMEM((CI, DIM), jnp.float32),
            pltpu.VMEM((CI, DIM), jnp.float32),
            pltpu.VMEM((CI, DIM), jnp.float32),
            pltpu.VMEM((RPW, DIM), jnp.float32),
            pltpu.VMEM((RPW,), jnp.int32),
            pltpu.VMEM((RPW, DIM), jnp.float32),
            pltpu.SemaphoreType.DMA,
            pltpu.SemaphoreType.DMA,
            pltpu.SemaphoreType.DMA,
            pltpu.SemaphoreType.DMA,
            pltpu.SemaphoreType.DMA,
        ],
    )
    def sc_kernel(s_hbm, o_hbm, p_hbm, ww_hbm, wr_hbm,
                  s_out, o_out, p_out,
                  idx_v, gbuf0, gbuf1, gbuf2, gbuf3, out_v, pidx_v, prow_v,
                  sem0, sem1, sem2, sem3, psem):
        wid = lax.axis_index("s") * NC + lax.axis_index("c")
        base = wid * RPW
        bufs = (gbuf0, gbuf1, gbuf2, gbuf3)
        sems = (sem0, sem1, sem2, sem3)
        nbuf = len(bufs)

        pltpu.sync_copy(p_hbm.at[pl.ds(base, RPW)], pidx_v)
        pltpu.make_async_copy(wr_hbm.at[pidx_v], prow_v, psem).start()

        def run_side(side_hbm, side_out):
            pltpu.sync_copy(side_hbm.at[wid], idx_v)

            def start(c, b):
                pltpu.make_async_copy(ww_hbm.at[idx_v.at[c]], bufs[b],
                                      sems[b]).start()

            def wait(b):
                pltpu.make_async_copy(ww_hbm.at[idx_v.at[0]], bufs[b],
                                      sems[b]).wait()

            def reduce_chunk(c, gbuf):
                for r in range(CR):
                    accs = [gbuf[r * L, pl.ds(d * 16, 16)] for d in range(NV)]
                    for l in range(1, L):
                        for d in range(NV):
                            accs[d] = accs[d] + gbuf[r * L + l,
                                                     pl.ds(d * 16, 16)]
                    row = c * CR + r
                    for d in range(NV):
                        out_v[row, pl.ds(d * 16, 16)] = accs[d]

            for b in range(nbuf - 1):
                start(b, b)

            def body(g, carry):
                for b in range(nbuf):
                    c = g * nbuf + b
                    nxt = c + nbuf - 1

                    @pl.when(nxt < NCH)
                    def _():
                        start(nxt, (b + nbuf - 1) % nbuf)

                    wait(b)
                    reduce_chunk(c, bufs[b])
                return carry

            lax.fori_loop(0, NCH // nbuf, body, 0)
            pltpu.sync_copy(out_v, side_out.at[pl.ds(base, RPW)])

        run_side(s_hbm, s_out)
        run_side(o_hbm, o_out)

        pltpu.make_async_copy(wr_hbm.at[pidx_v], prow_v, psem).wait()
        pltpu.sync_copy(prow_v, p_out.at[pl.ds(base, RPW)])

    return sc_kernel(s_r, o_r, p, W_words, W_rel)


def _tc_dense(s, o, s_sum, o_sum, p_emb, W_lin_t, b_lin):
    def tc_kernel(s_ref, o_ref, ssum_ref, osum_ref, pemb_ref,
                  wlt_ref, bl_ref, out_ref):
        freq_s = jnp.sum((s_ref[...] != 0).astype(jnp.float32), axis=1,
                         keepdims=True)
        freq_o = jnp.sum((o_ref[...] != 0).astype(jnp.float32), axis=1,
                         keepdims=True)
        se = ssum_ref[...] * freq_s
        oe = osum_ref[...] * freq_o
        wlt = wlt_ref[...]
        bl = bl_ref[...]
        st = jnp.maximum(
            jnp.dot(se, wlt, preferred_element_type=jnp.float32) + bl, 0.0)
        ot = jnp.maximum(
            jnp.dot(oe, wlt, preferred_element_type=jnp.float32) + bl, 0.0)
        out_ref[...] = jnp.sum(st * pemb_ref[...] * ot, axis=1)

    return pl.pallas_call(
        tc_kernel,
        out_shape=jax.ShapeDtypeStruct((B,), jnp.float32),
    )(s, o, s_sum, o_sum, p_emb, W_lin_t, b_lin)


def kernel(s, o, p, W_words, W_rel, W_lin, b_lin):
    s_r = s.reshape(NW, NCH, CI).astype(jnp.int32)
    o_r = o.reshape(NW, NCH, CI).astype(jnp.int32)
    p_i = p.astype(jnp.int32)
    s_sum, o_sum, p_emb = _sc_pool(s_r, o_r, p_i, W_words, W_rel)
    bl = b_lin.reshape(1, DIM)
    return _tc_dense(s, o, s_sum, o_sum, p_emb, W_lin.T, bl)
